# Initial kernel scaffold; baseline (speedup 1.0000x reference)
import jax
import jax.numpy as jnp
from jax.experimental import pallas as pl


def kernel(buffer, elements, mask, num_cells):
    raise NotImplementedError("write your pallas kernel here")


# speedup vs baseline: 2.1489x; 2.1489x over previous
import jax
import jax.numpy as jnp
from jax.experimental import pallas as pl
from jax.experimental.pallas import tpu as pltpu

_B = 640


def _scan_kernel(nblocks, m_ref, cx_ref, s0_ref, tot_ref, carry):
    g = pl.program_id(0)
    B = _B

    @pl.when(g == 0)
    def _():
        carry[0] = 0

    c0 = carry[0]
    m = m_ref[0]
    x = m
    sh = 1
    while sh < B:
        x = x + jnp.concatenate(
            [jnp.zeros((1, sh), jnp.int32), x[:, : B - sh]], axis=1
        )
        sh *= 2
    incl = x
    cx = c0 + incl - m
    cx_ref[0] = cx
    bsum = jnp.sum(m)

    d_start = g * B + c0
    d_end = (g + 1) * B + c0 + bsum
    iota = jax.lax.broadcasted_iota(jnp.int32, (1, B), 1)
    d_local = g * B + iota + cx

    g_lo = (d_start + B - 1) // B
    for k in range(3):
        gg = g_lo + k

        @pl.when((gg * B < d_end) & (gg < nblocks))
        def _():
            cnt = jnp.sum(jnp.where(d_local <= gg * B, 1, 0))
            s0_ref[gg] = g * B + cnt - 1

    @pl.when(g == nblocks - 1)
    def _():
        tot_ref[0] = c0 + bsum

    carry[0] = c0 + bsum


def _gather2(S, I):
    g0 = jnp.take_along_axis(S[:, :128], jnp.minimum(I, 127), axis=1)
    g1 = jnp.take_along_axis(S[:, 128:], jnp.maximum(I - 128, 0), axis=1)
    return jnp.where(I < 128, g0, g1)


def _gather_kernel(nvars, s0_ref, b0, b1, e0, e1, cx0, cx1, m0, out_ref,
                   winb, wine, dscr):
    g = pl.program_id(0)
    B = _B
    W = 2 * B
    V = nvars
    s0v = s0_ref[g]
    sbase = (s0v // B) * B

    winb[:, :B] = b0[...]
    winb[:, B:] = b1[...]
    wine[:, :B] = e0[...]
    wine[:, B:] = e1[...]

    cxw = jnp.concatenate([cx0[0], cx1[0]], axis=1)
    iota_w = jax.lax.broadcasted_iota(jnp.int32, (1, W), 1)
    d = sbase + iota_w + cxw
    dscr[...] = jnp.broadcast_to(d, (8, W))

    for k in range(B // 128):
        j0k = g * B + 128 * k
        jvec = j0k + jax.lax.broadcasted_iota(jnp.int32, (8, 128), 1)
        r = jnp.sum(jnp.where(d <= j0k, 1, 0)) - 1
        q = jnp.minimum(r // 128, W // 128 - 2)
        base = q * 128
        sub_d = dscr[:, pl.ds(base, 256)]

        lo = jnp.zeros((8, 128), jnp.int32)
        for step in (128, 64, 32, 16, 8, 4, 2, 1):
            cand = jnp.minimum(lo + step, 255)
            val = _gather2(sub_d, cand)
            lo = jnp.where(val <= jvec, cand, lo)

        dsrc = _gather2(sub_d, lo)
        is_ins = ((jvec - dsrc) == 1)[0:1]

        idx = jnp.broadcast_to(lo[0:1], (V, 128))
        outb = _gather2(winb[:, pl.ds(base, 256)], idx)
        oute = _gather2(wine[:, pl.ds(base, 256)], idx)
        res = jnp.where(is_ins, oute, outb)
        if k == 0:
            cond0 = (jvec[0:1] == 0) & (m0[0][:, 0:1] > 0)
            res = jnp.where(cond0, e0[:, 0:1], res)
        out_ref[:, 128 * k : 128 * (k + 1)] = res


def kernel(buffer, elements, mask, num_cells):
    V, N = buffer.shape
    B = _B
    assert N % B == 0, (N, B)
    G = N // B

    mask_i32 = mask.astype(jnp.int32).reshape(G, 1, B)

    cx, s0, tot = pl.pallas_call(
        lambda *refs: _scan_kernel(G, *refs),
        grid=(G,),
        in_specs=[pl.BlockSpec((1, 1, B), lambda g: (g, 0, 0))],
        out_specs=[
            pl.BlockSpec((1, 1, B), lambda g: (g, 0, 0)),
            pl.BlockSpec(memory_space=pltpu.SMEM),
            pl.BlockSpec(memory_space=pltpu.SMEM),
        ],
        out_shape=[
            jax.ShapeDtypeStruct((G, 1, B), jnp.int32),
            jax.ShapeDtypeStruct((G,), jnp.int32),
            jax.ShapeDtypeStruct((1,), jnp.int32),
        ],
        scratch_shapes=[pltpu.SMEM((1,), jnp.int32)],
    )(mask_i32)

    def blk0(g, s0r):
        return s0r[g] // B

    def blk1(g, s0r):
        return jnp.minimum(s0r[g] // B + 1, G - 1)

    grid_spec = pltpu.PrefetchScalarGridSpec(
        num_scalar_prefetch=1,
        grid=(G,),
        in_specs=[
            pl.BlockSpec((V, B), lambda g, s: (0, blk0(g, s))),
            pl.BlockSpec((V, B), lambda g, s: (0, blk1(g, s))),
            pl.BlockSpec((V, B), lambda g, s: (0, blk0(g, s))),
            pl.BlockSpec((V, B), lambda g, s: (0, blk1(g, s))),
            pl.BlockSpec((1, 1, B), lambda g, s: (blk0(g, s), 0, 0)),
            pl.BlockSpec((1, 1, B), lambda g, s: (blk1(g, s), 0, 0)),
            pl.BlockSpec((1, 1, B), lambda g, s: (blk0(g, s), 0, 0)),
        ],
        out_specs=pl.BlockSpec((V, B), lambda g, s: (0, g)),
        scratch_shapes=[
            pltpu.VMEM((V, 2 * _B), jnp.float32),
            pltpu.VMEM((V, 2 * _B), jnp.float32),
            pltpu.VMEM((8, 2 * _B), jnp.int32),
        ],
    )

    out = pl.pallas_call(
        lambda *refs: _gather_kernel(V, *refs),
        grid_spec=grid_spec,
        out_shape=jax.ShapeDtypeStruct((V, N), buffer.dtype),
    )(s0, buffer, buffer, elements, elements, cx, cx, mask_i32)

    return (out, num_cells + tot[0])


# speedup vs baseline: 2.7437x; 1.2768x over previous
import jax
import jax.numpy as jnp
from jax.experimental import pallas as pl
from jax.experimental.pallas import tpu as pltpu

_B = 640


_B1 = 3200


def _scan_kernel(nscan, nblocks, m_ref, cx_ref, s0_ref, tot_ref, carry):
    g = pl.program_id(0)
    B = _B
    B1 = _B1

    @pl.when(g == 0)
    def _():
        carry[0] = 0

    c0 = carry[0]
    m = m_ref[0]
    x = m
    sh = 1
    while sh < B1:
        x = x + jnp.concatenate(
            [jnp.zeros((1, sh), jnp.int32), x[:, : B1 - sh]], axis=1
        )
        sh *= 2
    incl = x
    cx = c0 + incl - m
    cx_ref[0] = cx
    bsum = jnp.sum(m)

    d_start = g * B1 + c0
    d_end = (g + 1) * B1 + c0 + bsum
    iota = jax.lax.broadcasted_iota(jnp.int32, (1, B1), 1)
    d_local = g * B1 + iota + cx

    g_lo = (d_start + B - 1) // B
    for k in range(2 * B1 // B + 1):
        gg = g_lo + k

        @pl.when((gg * B < d_end) & (gg < nblocks))
        def _():
            cnt = jnp.sum(jnp.where(d_local <= gg * B, 1, 0))
            s0_ref[gg] = g * B1 + cnt - 1

    @pl.when(g == nscan - 1)
    def _():
        tot_ref[0] = c0 + bsum

    carry[0] = c0 + bsum


def _gather2(S, I):
    g0 = jnp.take_along_axis(S[:, :128], jnp.minimum(I, 127), axis=1)
    g1 = jnp.take_along_axis(S[:, 128:], jnp.maximum(I - 128, 0), axis=1)
    return jnp.where(I < 128, g0, g1)


def _gather_kernel(nvars, s0_ref, b0, b1, e0, e1, cx0, cx1, m0, out_ref,
                   winb, wine, dscr):
    g = pl.program_id(0)
    B = _B
    W = 2 * B
    V = nvars
    s0v = s0_ref[g]
    sbase = (s0v // B) * B

    winb[:, :B] = b0[...]
    winb[:, B:] = b1[...]
    wine[:, :B] = e0[...]
    wine[:, B:] = e1[...]

    cxw = jnp.concatenate([cx0[0], cx1[0]], axis=1)
    iota_w = jax.lax.broadcasted_iota(jnp.int32, (1, W), 1)
    d = sbase + iota_w + cxw
    dscr[...] = jnp.broadcast_to(d, (8, W))

    for k in range(B // 128):
        j0k = g * B + 128 * k
        jvec = j0k + jax.lax.broadcasted_iota(jnp.int32, (8, 128), 1)
        r = jnp.sum(jnp.where(d <= j0k, 1, 0)) - 1
        q = jnp.minimum(r // 128, W // 128 - 2)
        base = q * 128
        sub_d = dscr[:, pl.ds(base, 256)]

        lo = jnp.zeros((8, 128), jnp.int32)
        for step in (128, 64, 32, 16, 8, 4, 2, 1):
            cand = jnp.minimum(lo + step, 255)
            val = _gather2(sub_d, cand)
            lo = jnp.where(val <= jvec, cand, lo)

        dsrc = _gather2(sub_d, lo)
        is_ins = ((jvec - dsrc) == 1)[0:1]

        idx = jnp.broadcast_to(lo[0:1], (V, 128))
        outb = _gather2(winb[:, pl.ds(base, 256)], idx)
        oute = _gather2(wine[:, pl.ds(base, 256)], idx)
        res = jnp.where(is_ins, oute, outb)
        if k == 0:
            cond0 = (jvec[0:1] == 0) & (m0[0][:, 0:1] > 0)
            res = jnp.where(cond0, e0[:, 0:1], res)
        out_ref[:, 128 * k : 128 * (k + 1)] = res


def kernel(buffer, elements, mask, num_cells):
    V, N = buffer.shape
    B = _B
    assert N % B == 0, (N, B)
    G = N // B

    B1 = _B1
    assert N % B1 == 0, (N, B1)
    G1 = N // B1
    mask_cast = mask.astype(jnp.int32)
    mask_i32 = mask_cast.reshape(G, 1, B)

    cx, s0, tot = pl.pallas_call(
        lambda *refs: _scan_kernel(G1, G, *refs),
        grid=(G1,),
        in_specs=[pl.BlockSpec((1, 1, B1), lambda g: (g, 0, 0))],
        out_specs=[
            pl.BlockSpec((1, 1, B1), lambda g: (g, 0, 0)),
            pl.BlockSpec(memory_space=pltpu.SMEM),
            pl.BlockSpec(memory_space=pltpu.SMEM),
        ],
        out_shape=[
            jax.ShapeDtypeStruct((G1, 1, B1), jnp.int32),
            jax.ShapeDtypeStruct((G,), jnp.int32),
            jax.ShapeDtypeStruct((1,), jnp.int32),
        ],
        scratch_shapes=[pltpu.SMEM((1,), jnp.int32)],
    )(mask_cast.reshape(G1, 1, B1))
    cx = cx.reshape(G, 1, B)

    def blk0(g, s0r):
        return s0r[g] // B

    def blk1(g, s0r):
        return jnp.minimum(s0r[g] // B + 1, G - 1)

    grid_spec = pltpu.PrefetchScalarGridSpec(
        num_scalar_prefetch=1,
        grid=(G,),
        in_specs=[
            pl.BlockSpec((V, B), lambda g, s: (0, blk0(g, s))),
            pl.BlockSpec((V, B), lambda g, s: (0, blk1(g, s))),
            pl.BlockSpec((V, B), lambda g, s: (0, blk0(g, s))),
            pl.BlockSpec((V, B), lambda g, s: (0, blk1(g, s))),
            pl.BlockSpec((1, 1, B), lambda g, s: (blk0(g, s), 0, 0)),
            pl.BlockSpec((1, 1, B), lambda g, s: (blk1(g, s), 0, 0)),
            pl.BlockSpec((1, 1, B), lambda g, s: (blk0(g, s), 0, 0)),
        ],
        out_specs=pl.BlockSpec((V, B), lambda g, s: (0, g)),
        scratch_shapes=[
            pltpu.VMEM((V, 2 * _B), jnp.float32),
            pltpu.VMEM((V, 2 * _B), jnp.float32),
            pltpu.VMEM((8, 2 * _B), jnp.int32),
        ],
    )

    out = pl.pallas_call(
        lambda *refs: _gather_kernel(V, *refs),
        grid_spec=grid_spec,
        out_shape=jax.ShapeDtypeStruct((V, N), buffer.dtype),
    )(s0, buffer, buffer, elements, elements, cx, cx, mask_i32)

    return (out, num_cells + tot[0])
